# Initial kernel scaffold; baseline (speedup 1.0000x reference)
import jax
import jax.numpy as jnp
from jax.experimental import pallas as pl


def kernel(z, codebook):
    raise NotImplementedError("write your pallas kernel here")


# speedup vs baseline: 1.0417x; 1.0417x over previous
import functools

import jax
import jax.numpy as jnp
from jax import lax
from jax.experimental import pallas as pl
from jax.experimental.pallas import tpu as pltpu
from jax.experimental.pallas import tpu_sc as plsc

_K = 8192
_D = 256
_N = 8192
_NB = 512
_KB = 1024
_NN = _N // _NB
_NK = _K // _KB

_SEG_BOUNDS = (2048, 4096, 6144)

_COMMIT_COST = 0.25


def _dist_body(z_ref, cb_ref, enc_ref, loss_ref, zsq_ref, minv_ref, mini_ref):
    n = pl.program_id(0)
    k = pl.program_id(1)
    z = z_ref[...]
    cb = cb_ref[...]

    @pl.when(k == 0)
    def _init():
        zsq_ref[...] = jnp.sum(z * z, axis=1, keepdims=True)
        minv_ref[...] = jnp.full((_NB, 1), jnp.inf, jnp.float32)
        mini_ref[...] = jnp.zeros((_NB, 1), jnp.int32)

    esq = jnp.sum(cb * cb, axis=1)
    z_bf = z.astype(jnp.bfloat16)
    cb_bf = cb.astype(jnp.bfloat16)
    mm = lax.dot_general(z_bf, cb_bf, (((1,), (1,)), ((), ())),
                         preferred_element_type=jnp.float32)
    dist = (zsq_ref[...] + esq[None, :]) - 2.0 * mm

    gids = lax.broadcasted_iota(jnp.int32, (_NB, _KB), 1) + k * _KB

    def _fold(mask):
        dsel = jnp.where(mask, dist, jnp.inf)
        lmin = jnp.min(dsel, axis=1, keepdims=True)
        larg = jnp.min(jnp.where(mask & (dist == lmin), gids, _K),
                       axis=1, keepdims=True)
        better = lmin < minv_ref[...]
        mini_ref[...] = jnp.where(better, larg, mini_ref[...])
        minv_ref[...] = jnp.where(better, lmin, minv_ref[...])

    def _round_acc():
        minv_ref[...] = minv_ref[...].astype(jnp.bfloat16).astype(jnp.float32)

    at_seg_start = (k == _SEG_BOUNDS[0] // _KB)
    for _b in _SEG_BOUNDS[1:]:
        at_seg_start = jnp.logical_or(at_seg_start, k == _b // _KB)

    @pl.when(at_seg_start)
    def _round():
        _round_acc()

    _fold(jnp.full((_NB, _KB), True))

    @pl.when(k == _NK - 1)
    def _finish():
        enc_ref[...] = mini_ref[...]
        psum = jnp.sum(minv_ref[...])

        @pl.when(n == 0)
        def _():
            loss_ref[0, 0] = psum

        @pl.when(n != 0)
        def _():
            loss_ref[0, 0] = loss_ref[0, 0] + psum


_dist_call = pl.pallas_call(
    _dist_body,
    grid=(_NN, _NK),
    in_specs=[
        pl.BlockSpec((_NB, _D), lambda n, k: (n, 0)),
        pl.BlockSpec((_KB, _D), lambda n, k: (k, 0)),
    ],
    out_specs=[
        pl.BlockSpec((_NB, 1), lambda n, k: (n, 0)),
        pl.BlockSpec(memory_space=pltpu.SMEM),
    ],
    out_shape=[
        jax.ShapeDtypeStruct((_N, 1), jnp.int32),
        jax.ShapeDtypeStruct((1, 1), jnp.float32),
    ],
    scratch_shapes=[
        pltpu.VMEM((_NB, 1), jnp.float32),
        pltpu.VMEM((_NB, 1), jnp.float32),
        pltpu.VMEM((_NB, 1), jnp.int32),
    ],
)


_NC = 2
_NS = 16
_NW = _NC * _NS
_BPW = _N // _NW
_CHUNK = 128
_NCH = _BPW // _CHUNK


@functools.cache
def _make_sc_gather():
    mesh = plsc.VectorSubcoreMesh(core_axis_name="c", subcore_axis_name="s")

    @functools.partial(
        pl.kernel,
        mesh=mesh,
        out_type=jax.ShapeDtypeStruct((_N, _D), jnp.float32),
        scratch_types=[
            pltpu.VMEM((_NCH, _CHUNK), jnp.int32),
            pltpu.VMEM((_BPW, _D), jnp.float32),
            pltpu.SemaphoreType.DMA,
        ],
    )
    def _sc_gather(cb_hbm, idx_hbm, out_hbm, idx_v, rows_v, sem):
        wid = lax.axis_index("s") * _NC + lax.axis_index("c")
        base = wid * _BPW
        pltpu.sync_copy(idx_hbm.at[pl.ds(wid * _NCH, _NCH)], idx_v)
        copies = [
            pltpu.async_copy(cb_hbm.at[idx_v.at[j]],
                             rows_v.at[pl.ds(j * _CHUNK, _CHUNK)], sem)
            for j in range(_NCH)
        ]
        for c in copies:
            c.wait()
        pltpu.sync_copy(rows_v, out_hbm.at[pl.ds(base, _BPW)])

    return _sc_gather


def kernel(z, codebook):
    flat_z = z.reshape(-1, _D)
    enc2d, loss_sum = _dist_call(flat_z, codebook)
    encoding = enc2d.reshape(-1)
    code = _make_sc_gather()(codebook, encoding.reshape(_N // _CHUNK, _CHUNK))
    code = code.astype(jnp.bfloat16).astype(jnp.float32)
    code_st = code.reshape(z.shape)
    commitment_loss = (_COMMIT_COST / (_N * _D)) * loss_sum[0, 0]
    return (code_st, commitment_loss, None, encoding)


# speedup vs baseline: 1.1115x; 1.0669x over previous
import functools

import jax
import jax.numpy as jnp
from jax import lax
from jax.experimental import pallas as pl
from jax.experimental.pallas import tpu as pltpu
from jax.experimental.pallas import tpu_sc as plsc

_K = 8192
_D = 256
_N = 8192
_NB = 512
_KB = 2048
_NN = _N // _NB
_NK = _K // _KB

_SEG_BOUNDS = (2048, 4096, 6144)

_COMMIT_COST = 0.25


def _dist_body(z_ref, cb_ref, enc_ref, loss_ref, zsq_ref, minv_ref, mini_ref):
    k = pl.program_id(0)
    n = pl.program_id(1)
    z = z_ref[...]
    cb = cb_ref[...]
    rows = pl.ds(n * _NB, _NB)

    @pl.when(k == 0)
    def _init():
        zsq_ref[rows, :] = jnp.sum(z * z, axis=1, keepdims=True)
        minv_ref[rows, :] = jnp.full((_NB, 1), jnp.inf, jnp.float32)
        mini_ref[rows, :] = jnp.zeros((_NB, 1), jnp.int32)

    esq = jnp.sum(cb * cb, axis=1)
    z_bf = z.astype(jnp.bfloat16)
    cb_bf = cb.astype(jnp.bfloat16)
    mm = lax.dot_general(z_bf, cb_bf, (((1,), (1,)), ((), ())),
                         preferred_element_type=jnp.float32)
    dist = (zsq_ref[rows, :] + esq[None, :]) - 2.0 * mm

    gids = lax.broadcasted_iota(jnp.int32, (_NB, _KB), 1) + k * _KB

    lmin = jnp.min(dist, axis=1, keepdims=True)
    larg = jnp.min(jnp.where(dist == lmin, gids, _K), axis=1, keepdims=True)

    acc = minv_ref[rows, :]

    @pl.when(k != 0)
    def _cmp():
        accr = acc.astype(jnp.bfloat16).astype(jnp.float32)
        better = lmin < accr
        mini_ref[rows, :] = jnp.where(better, larg, mini_ref[rows, :])
        minv_ref[rows, :] = jnp.where(better, lmin, accr)

    @pl.when(k == 0)
    def _first():
        mini_ref[rows, :] = larg
        minv_ref[rows, :] = lmin

    @pl.when(k == _NK - 1)
    def _finish():
        enc_ref[...] = mini_ref[rows, :]
        psum = jnp.sum(minv_ref[rows, :])

        @pl.when(n == 0)
        def _():
            loss_ref[0, 0] = psum

        @pl.when(n != 0)
        def _():
            loss_ref[0, 0] = loss_ref[0, 0] + psum


_dist_call = pl.pallas_call(
    _dist_body,
    grid=(_NK, _NN),
    in_specs=[
        pl.BlockSpec((_NB, _D), lambda k, n: (n, 0)),
        pl.BlockSpec((_KB, _D), lambda k, n: (k, 0)),
    ],
    out_specs=[
        pl.BlockSpec((_NB, 1), lambda k, n: (n, 0)),
        pl.BlockSpec(memory_space=pltpu.SMEM),
    ],
    out_shape=[
        jax.ShapeDtypeStruct((_N, 1), jnp.int32),
        jax.ShapeDtypeStruct((1, 1), jnp.float32),
    ],
    scratch_shapes=[
        pltpu.VMEM((_N, 1), jnp.float32),
        pltpu.VMEM((_N, 1), jnp.float32),
        pltpu.VMEM((_N, 1), jnp.int32),
    ],
)


_NC = 2
_NS = 16
_NW = _NC * _NS
_BPW = _N // _NW
_CHUNK = 128
_NCH = _BPW // _CHUNK


@functools.cache
def _make_sc_gather():
    mesh = plsc.VectorSubcoreMesh(core_axis_name="c", subcore_axis_name="s")

    @functools.partial(
        pl.kernel,
        mesh=mesh,
        out_type=jax.ShapeDtypeStruct((_N, _D), jnp.float32),
        scratch_types=[
            pltpu.VMEM((_NCH, _CHUNK), jnp.int32),
            pltpu.VMEM((_BPW, _D), jnp.float32),
            pltpu.SemaphoreType.DMA,
        ],
    )
    def _sc_gather(cb_hbm, idx_hbm, out_hbm, idx_v, rows_v, sem):
        wid = lax.axis_index("s") * _NC + lax.axis_index("c")
        base = wid * _BPW
        pltpu.sync_copy(idx_hbm.at[pl.ds(wid * _NCH, _NCH)], idx_v)
        copies = [
            pltpu.async_copy(cb_hbm.at[idx_v.at[j]],
                             rows_v.at[pl.ds(j * _CHUNK, _CHUNK)], sem)
            for j in range(_NCH)
        ]
        for c in copies:
            c.wait()
        pltpu.sync_copy(rows_v, out_hbm.at[pl.ds(base, _BPW)])

    return _sc_gather


def kernel(z, codebook):
    flat_z = z.reshape(-1, _D)
    enc2d, loss_sum = _dist_call(flat_z, codebook)
    encoding = enc2d.reshape(-1)
    code = _make_sc_gather()(codebook, encoding.reshape(_N // _CHUNK, _CHUNK))
    code = code.astype(jnp.bfloat16).astype(jnp.float32)
    code_st = code.reshape(z.shape)
    commitment_loss = (_COMMIT_COST / (_N * _D)) * loss_sum[0, 0]
    return (code_st, commitment_loss, None, encoding)
